# Initial kernel scaffold; baseline (speedup 1.0000x reference)
import jax
import jax.numpy as jnp
from jax.experimental import pallas as pl


def kernel(state, context, action, v, dirs):
    raise NotImplementedError("write your pallas kernel here")


# speedup vs baseline: 1.8515x; 1.8515x over previous
import functools

import jax
import jax.numpy as jnp
from jax import lax
from jax.experimental import pallas as pl
from jax.experimental.pallas import tpu as pltpu
from jax.experimental.pallas import tpu_sc as plsc

B = 16384
NC = 2
NS = 16
NW = NC * NS
BPW = B // NW
LANES = 16


def _make_gather_kernel():
    mesh = plsc.VectorSubcoreMesh(core_axis_name="c", subcore_axis_name="s")

    @functools.partial(
        pl.kernel,
        mesh=mesh,
        compiler_params=pltpu.CompilerParams(needs_layout_passes=False),
        out_type=jax.ShapeDtypeStruct((B * 2,), jnp.float32),
        scratch_types=[
            pltpu.VMEM((LANES,), jnp.float32),
            pltpu.VMEM((LANES,), jnp.float32),
            pltpu.VMEM((BPW,), jnp.int32),
            pltpu.VMEM((BPW * 2,), jnp.float32),
        ],
    )
    def gather_add(dirs_hbm, vpat_hbm, act_hbm, out_hbm, tab_v, vv_v, idx_v, out_v):
        wid = lax.axis_index("s") * NC + lax.axis_index("c")
        base = wid * BPW
        pltpu.sync_copy(dirs_hbm, tab_v)
        pltpu.sync_copy(vpat_hbm, vv_v)
        pltpu.sync_copy(act_hbm.at[pl.ds(base, BPW)], idx_v)
        tab_v[...] = tab_v[...] + vv_v[...]
        lane = lax.iota(jnp.int32, LANES)
        for k in range(BPW // LANES):
            a = idx_v[pl.ds(k * LANES, LANES)]
            i0 = a * 2
            g0 = plsc.load_gather(tab_v, [i0])
            g1 = plsc.load_gather(tab_v, [i0 + 1])
            pos = k * (2 * LANES) + lane * 2
            plsc.store_scatter(out_v, [pos], g0)
            plsc.store_scatter(out_v, [pos + 1], g1)
        pltpu.sync_copy(out_v, out_hbm.at[pl.ds(base * 2, BPW * 2)])

    return gather_add


_gather_add = _make_gather_kernel()


def kernel(state, context, action, v, dirs):
    dirs_pad = jnp.pad(dirs.reshape(-1), (0, LANES - 2 * dirs.shape[0]))
    vpat = jnp.tile(v, LANES // 2)
    act_flat = action.reshape(B)
    delta_flat = _gather_add(dirs_pad, vpat, act_flat)
    delta = delta_flat.reshape(B, 1, 2)
    halves = jnp.full((B, 1, 2), 0.5, dtype=jnp.float32)
    weights = jnp.ones((B, 1), dtype=jnp.float32)
    resid = jnp.zeros((B,), dtype=jnp.float32)
    return ((delta, halves), weights, resid)


# speedup vs baseline: 3.0049x; 1.6230x over previous
import functools

import jax
import jax.numpy as jnp
from jax import lax
from jax.experimental import pallas as pl
from jax.experimental.pallas import tpu as pltpu
from jax.experimental.pallas import tpu_sc as plsc

B = 16384
NC = 2
NS = 16
NW = NC * NS
BPW = B // NW
LANES = 16
NBLK = B // 128
BLK_PER_W = BPW // 128


def _make_gather_kernel():
    mesh = plsc.VectorSubcoreMesh(core_axis_name="c", subcore_axis_name="s")

    @functools.partial(
        pl.kernel,
        mesh=mesh,
        compiler_params=pltpu.CompilerParams(needs_layout_passes=False),
        out_type=jax.ShapeDtypeStruct((NBLK, 2, 128), jnp.float32),
        scratch_types=[
            pltpu.VMEM((4, 2), jnp.float32),
            pltpu.VMEM((2,), jnp.float32),
            pltpu.VMEM((LANES,), jnp.float32),
            pltpu.VMEM((BPW,), jnp.int32),
            pltpu.VMEM((BLK_PER_W, 2, 128), jnp.float32),
        ],
    )
    def gather_add(dirs_hbm, v_hbm, act_hbm, out_hbm,
                   dirs_v, vv_v, tab_v, idx_v, out_v):
        wid = lax.axis_index("s") * NC + lax.axis_index("c")
        base = wid * BPW
        pltpu.sync_copy(dirs_hbm, dirs_v)
        pltpu.sync_copy(v_hbm, vv_v)
        pltpu.sync_copy(act_hbm.at[pl.ds(base, BPW)], idx_v)
        lane = lax.iota(jnp.int32, LANES)
        row = lane // 2
        col = lane - row * 2
        tab_v[...] = (plsc.load_gather(dirs_v, [row, col])
                      + plsc.load_gather(vv_v, [col]))
        for k in range(BPW // LANES):
            a = idx_v[pl.ds(k * LANES, LANES)]
            i0 = a * 2
            g0 = plsc.load_gather(tab_v, [i0])
            g1 = plsc.load_gather(tab_v, [i0 + 1])
            t, off = divmod(k * LANES, 128)
            out_v[t, 0, pl.ds(off, LANES)] = g0
            out_v[t, 1, pl.ds(off, LANES)] = g1
        pltpu.sync_copy(out_v, out_hbm.at[pl.ds(wid * BLK_PER_W, BLK_PER_W)])

    return gather_add


_gather_add = _make_gather_kernel()


def kernel(state, context, action, v, dirs):
    act_flat = action.reshape(B)
    out_blocks = _gather_add(dirs, v, act_flat)
    delta = out_blocks.transpose(0, 2, 1).reshape(B, 1, 2)
    halves = jnp.full((B, 1, 2), 0.5, dtype=jnp.float32)
    weights = jnp.ones((B, 1), dtype=jnp.float32)
    resid = jnp.zeros((B,), dtype=jnp.float32)
    return ((delta, halves), weights, resid)


# speedup vs baseline: 3.1988x; 1.0645x over previous
import functools

import jax
import jax.numpy as jnp
from jax import lax
from jax.experimental import pallas as pl
from jax.experimental.pallas import tpu as pltpu
from jax.experimental.pallas import tpu_sc as plsc

B = 16384
NC = 2
NS = 16
NW = NC * NS
BPW = B // NW
LANES = 16
NBLK = B // 128
BLK_PER_W = BPW // 128


def _make_gather_kernel():
    mesh = plsc.VectorSubcoreMesh(core_axis_name="c", subcore_axis_name="s")

    @functools.partial(
        pl.kernel,
        mesh=mesh,
        compiler_params=pltpu.CompilerParams(needs_layout_passes=False),
        out_type=[
            jax.ShapeDtypeStruct((NBLK, 2, 128), jnp.float32),
            jax.ShapeDtypeStruct((NBLK, 2, 128), jnp.float32),
            jax.ShapeDtypeStruct((B,), jnp.float32),
            jax.ShapeDtypeStruct((B,), jnp.float32),
        ],
        scratch_types=[
            pltpu.VMEM((4, 2), jnp.float32),
            pltpu.VMEM((2,), jnp.float32),
            pltpu.VMEM((LANES,), jnp.float32),
            pltpu.VMEM((BPW,), jnp.int32),
            pltpu.VMEM((BLK_PER_W, 2, 128), jnp.float32),
            pltpu.VMEM((BLK_PER_W, 2, 128), jnp.float32),
            pltpu.VMEM((BPW,), jnp.float32),
            pltpu.VMEM((BPW,), jnp.float32),
        ],
    )
    def gather_add(dirs_hbm, v_hbm, act_hbm,
                   out_hbm, half_hbm, ones_hbm, zero_hbm,
                   dirs_v, vv_v, tab_v, idx_v, out_v, half_v, ones_v, zero_v):
        wid = lax.axis_index("s") * NC + lax.axis_index("c")
        base = wid * BPW
        pltpu.sync_copy(dirs_hbm, dirs_v)
        pltpu.sync_copy(v_hbm, vv_v)
        pltpu.sync_copy(act_hbm.at[pl.ds(base, BPW)], idx_v)
        lane = lax.iota(jnp.int32, LANES)
        row = lane // 2
        col = lane - row * 2
        tab_v[...] = (plsc.load_gather(dirs_v, [row, col])
                      + plsc.load_gather(vv_v, [col]))
        halves = jnp.full((LANES,), 0.5, dtype=jnp.float32)
        ones = jnp.full((LANES,), 1.0, dtype=jnp.float32)
        zeros = jnp.zeros((LANES,), dtype=jnp.float32)
        for k in range(BPW // LANES):
            a = idx_v[pl.ds(k * LANES, LANES)]
            i0 = a * 2
            g0 = plsc.load_gather(tab_v, [i0])
            g1 = plsc.load_gather(tab_v, [i0 + 1])
            t, off = divmod(k * LANES, 128)
            out_v[t, 0, pl.ds(off, LANES)] = g0
            out_v[t, 1, pl.ds(off, LANES)] = g1
            half_v[t, 0, pl.ds(off, LANES)] = halves
            half_v[t, 1, pl.ds(off, LANES)] = halves
            ones_v[pl.ds(k * LANES, LANES)] = ones
            zero_v[pl.ds(k * LANES, LANES)] = zeros
        pltpu.sync_copy(out_v, out_hbm.at[pl.ds(wid * BLK_PER_W, BLK_PER_W)])
        pltpu.sync_copy(half_v, half_hbm.at[pl.ds(wid * BLK_PER_W, BLK_PER_W)])
        pltpu.sync_copy(ones_v, ones_hbm.at[pl.ds(base, BPW)])
        pltpu.sync_copy(zero_v, zero_hbm.at[pl.ds(base, BPW)])

    return gather_add


_gather_add = _make_gather_kernel()


def kernel(state, context, action, v, dirs):
    act_flat = action.reshape(B)
    out_blocks, half2, ones1, zero1 = _gather_add(dirs, v, act_flat)
    delta = out_blocks.transpose(0, 2, 1).reshape(B, 1, 2)
    halves = half2.transpose(0, 2, 1).reshape(B, 1, 2)
    weights = ones1.reshape(B, 1)
    resid = zero1
    return ((delta, halves), weights, resid)


# speedup vs baseline: 3.3142x; 1.0361x over previous
import functools

import jax
import jax.numpy as jnp
from jax import lax
from jax.experimental import pallas as pl
from jax.experimental.pallas import tpu as pltpu
from jax.experimental.pallas import tpu_sc as plsc

B = 16384
NC = 2
NS = 16
NW = NC * NS
BPW = B // NW
LANES = 16
NBLK = B // 128
BLK_PER_W = BPW // 128


def _make_gather_kernel():
    mesh = plsc.VectorSubcoreMesh(core_axis_name="c", subcore_axis_name="s")

    @functools.partial(
        pl.kernel,
        mesh=mesh,
        compiler_params=pltpu.CompilerParams(needs_layout_passes=False),
        out_type=[
            jax.ShapeDtypeStruct((NBLK, 2, 128), jnp.float32),
            jax.ShapeDtypeStruct((NBLK, 2, 128), jnp.float32),
            jax.ShapeDtypeStruct((B,), jnp.float32),
            jax.ShapeDtypeStruct((B,), jnp.float32),
        ],
        scratch_types=[
            pltpu.VMEM((4, 2), jnp.float32),
            pltpu.VMEM((2,), jnp.float32),
            pltpu.VMEM((LANES,), jnp.float32),
            pltpu.VMEM((BPW,), jnp.int32),
            pltpu.VMEM((BLK_PER_W, 2, 128), jnp.float32),
            pltpu.VMEM((BLK_PER_W, 2, 128), jnp.float32),
            pltpu.VMEM((BPW,), jnp.float32),
            pltpu.VMEM((BPW,), jnp.float32),
        ],
    )
    def gather_add(dirs_hbm, v_hbm, act_hbm,
                   out_hbm, half_hbm, ones_hbm, zero_hbm,
                   dirs_v, vv_v, tab_v, idx_v, out_v, half_v, ones_v, zero_v):
        wid = lax.axis_index("s") * NC + lax.axis_index("c")
        base = wid * BPW
        pltpu.sync_copy(dirs_hbm, dirs_v)
        pltpu.sync_copy(v_hbm, vv_v)
        pltpu.sync_copy(act_hbm.at[pl.ds(base, BPW)], idx_v)
        lane = lax.iota(jnp.int32, LANES)
        row = lane // 2
        col = lane - row * 2
        tab_v[...] = (plsc.load_gather(dirs_v, [row, col])
                      + plsc.load_gather(vv_v, [col]))
        halves = jnp.full((LANES,), 0.5, dtype=jnp.float32)
        ones = jnp.full((LANES,), 1.0, dtype=jnp.float32)
        zeros = jnp.zeros((LANES,), dtype=jnp.float32)
        def body(t, _):
            for u in range(8):
                k = t * 8 + u
                off = u * LANES
                a = idx_v[pl.ds(k * LANES, LANES)]
                i0 = a * 2
                g0 = plsc.load_gather(tab_v, [i0])
                g1 = plsc.load_gather(tab_v, [i0 + 1])
                out_v[t, 0, pl.ds(off, LANES)] = g0
                out_v[t, 1, pl.ds(off, LANES)] = g1
                half_v[t, 0, pl.ds(off, LANES)] = halves
                half_v[t, 1, pl.ds(off, LANES)] = halves
                ones_v[pl.ds(k * LANES, LANES)] = ones
                zero_v[pl.ds(k * LANES, LANES)] = zeros
            return 0

        lax.fori_loop(0, BLK_PER_W, body, 0)
        pltpu.sync_copy(out_v, out_hbm.at[pl.ds(wid * BLK_PER_W, BLK_PER_W)])
        pltpu.sync_copy(half_v, half_hbm.at[pl.ds(wid * BLK_PER_W, BLK_PER_W)])
        pltpu.sync_copy(ones_v, ones_hbm.at[pl.ds(base, BPW)])
        pltpu.sync_copy(zero_v, zero_hbm.at[pl.ds(base, BPW)])

    return gather_add


_gather_add = _make_gather_kernel()


def kernel(state, context, action, v, dirs):
    act_flat = action.reshape(B)
    out_blocks, half2, ones1, zero1 = _gather_add(dirs, v, act_flat)
    delta = out_blocks.transpose(0, 2, 1).reshape(B, 1, 2)
    halves = half2.transpose(0, 2, 1).reshape(B, 1, 2)
    weights = ones1.reshape(B, 1)
    resid = zero1
    return ((delta, halves), weights, resid)


# speedup vs baseline: 3.4262x; 1.0338x over previous
import functools

import jax
import jax.numpy as jnp
from jax import lax
from jax.experimental import pallas as pl
from jax.experimental.pallas import tpu as pltpu
from jax.experimental.pallas import tpu_sc as plsc

B = 16384
NC = 2
NS = 16
NW = NC * NS
BPW = B // NW
LANES = 16
NBLK = B // 128
BLK_PER_W = BPW // 128


def _make_gather_kernel():
    mesh = plsc.VectorSubcoreMesh(core_axis_name="c", subcore_axis_name="s")

    @functools.partial(
        pl.kernel,
        mesh=mesh,
        compiler_params=pltpu.CompilerParams(needs_layout_passes=False),
        out_type=[
            jax.ShapeDtypeStruct((NBLK, 2, 128), jnp.float32),
            jax.ShapeDtypeStruct((NBLK, 2, 128), jnp.float32),
            jax.ShapeDtypeStruct((B,), jnp.float32),
            jax.ShapeDtypeStruct((B,), jnp.float32),
        ],
        scratch_types=[
            pltpu.VMEM((2, 4), jnp.float32),
            pltpu.VMEM((2,), jnp.float32),
            pltpu.VMEM((LANES,), jnp.float32),
            pltpu.VMEM((BPW,), jnp.int32),
            pltpu.VMEM((BLK_PER_W, 2, 128), jnp.float32),
            pltpu.VMEM((BLK_PER_W, 2, 128), jnp.float32),
            pltpu.VMEM((BPW,), jnp.float32),
            pltpu.VMEM((BPW,), jnp.float32),
            pltpu.SemaphoreType.DMA,
            pltpu.SemaphoreType.DMA,
        ],
    )
    def gather_add(dirs_hbm, v_hbm, act_hbm,
                   out_hbm, half_hbm, ones_hbm, zero_hbm,
                   dirs_v, vv_v, tab_v, idx_v, out_v, half_v, ones_v, zero_v,
                   sem_in, sem_out):
        wid = lax.axis_index("s") * NC + lax.axis_index("c")
        base = wid * BPW
        h_act = pltpu.async_copy(act_hbm.at[pl.ds(base, BPW)], idx_v, sem_in)
        h_dirs = pltpu.async_copy(dirs_hbm, dirs_v, sem_in)
        h_v = pltpu.async_copy(v_hbm, vv_v, sem_in)
        halves = jnp.full((LANES,), 0.5, dtype=jnp.float32)
        ones = jnp.full((LANES,), 1.0, dtype=jnp.float32)
        zeros = jnp.zeros((LANES,), dtype=jnp.float32)
        for k in range(BPW // LANES):
            t, off = divmod(k * LANES, 128)
            half_v[t, 0, pl.ds(off, LANES)] = halves
            half_v[t, 1, pl.ds(off, LANES)] = halves
            ones_v[pl.ds(k * LANES, LANES)] = ones
            zero_v[pl.ds(k * LANES, LANES)] = zeros
        blk = pl.ds(wid * BLK_PER_W, BLK_PER_W)
        h_half = pltpu.async_copy(half_v, half_hbm.at[blk], sem_out)
        h_ones = pltpu.async_copy(ones_v, ones_hbm.at[pl.ds(base, BPW)], sem_out)
        h_zero = pltpu.async_copy(zero_v, zero_hbm.at[pl.ds(base, BPW)], sem_out)
        h_dirs.wait()
        h_v.wait()
        lane = lax.iota(jnp.int32, LANES)
        row = lane // 2
        col = lane - row * 2
        tab_v[...] = (plsc.load_gather(dirs_v, [col, row])
                      + plsc.load_gather(vv_v, [col]))
        h_act.wait()

        def body(t, _):
            for u in range(8):
                k = t * 8 + u
                off = u * LANES
                a = idx_v[pl.ds(k * LANES, LANES)]
                i0 = a * 2
                g0 = plsc.load_gather(tab_v, [i0])
                g1 = plsc.load_gather(tab_v, [i0 + 1])
                out_v[t, 0, pl.ds(off, LANES)] = g0
                out_v[t, 1, pl.ds(off, LANES)] = g1
            return 0

        lax.fori_loop(0, BLK_PER_W, body, 0)
        h_out = pltpu.async_copy(out_v, out_hbm.at[blk], sem_out)
        h_half.wait()
        h_ones.wait()
        h_zero.wait()
        h_out.wait()

    return gather_add


_gather_add = _make_gather_kernel()


def kernel(state, context, action, v, dirs):
    act_flat = action.reshape(B)
    out_blocks, half2, ones1, zero1 = _gather_add(dirs.T, v, act_flat)
    delta = out_blocks.transpose(0, 2, 1).reshape(B, 1, 2)
    halves = half2.transpose(0, 2, 1).reshape(B, 1, 2)
    weights = ones1.reshape(B, 1)
    resid = zero1
    return ((delta, halves), weights, resid)


# speedup vs baseline: 3.4283x; 1.0006x over previous
import functools

import jax
import jax.numpy as jnp
from jax import lax
from jax.experimental import pallas as pl
from jax.experimental.pallas import tpu as pltpu
from jax.experimental.pallas import tpu_sc as plsc

B = 16384
NC = 2
NS = 16
NW = NC * NS
BPW = B // NW
LANES = 16
NBLK = B // 128
BLK_PER_W = BPW // 128


def _make_gather_kernel():
    mesh = plsc.VectorSubcoreMesh(core_axis_name="c", subcore_axis_name="s")

    @functools.partial(
        pl.kernel,
        mesh=mesh,
        compiler_params=pltpu.CompilerParams(needs_layout_passes=False),
        out_type=[
            jax.ShapeDtypeStruct((NBLK, 2, 128), jnp.float32),
            jax.ShapeDtypeStruct((NBLK, 2, 128), jnp.float32),
            jax.ShapeDtypeStruct((B,), jnp.float32),
            jax.ShapeDtypeStruct((B,), jnp.float32),
        ],
        scratch_types=[
            pltpu.VMEM((2, 4), jnp.float32),
            pltpu.VMEM((2,), jnp.float32),
            pltpu.VMEM((LANES,), jnp.float32),
            pltpu.VMEM((BPW,), jnp.int32),
            pltpu.VMEM((BLK_PER_W, 2, 128), jnp.float32),
            pltpu.VMEM((BLK_PER_W, 2, 128), jnp.float32),
            pltpu.VMEM((BPW,), jnp.float32),
            pltpu.VMEM((BPW,), jnp.float32),
            pltpu.SemaphoreType.DMA,
            pltpu.SemaphoreType.DMA,
        ],
    )
    def gather_add(dirs_hbm, v_hbm, act_hbm,
                   out_hbm, half_hbm, ones_hbm, zero_hbm,
                   dirs_v, vv_v, tab_v, idx_v, out_v, half_v, ones_v, zero_v,
                   sem_in, sem_out):
        wid = lax.axis_index("s") * NC + lax.axis_index("c")
        base = wid * BPW
        h_act = pltpu.async_copy(act_hbm.at[pl.ds(base, BPW)], idx_v, sem_in)
        h_dirs = pltpu.async_copy(dirs_hbm, dirs_v, sem_in)
        h_v = pltpu.async_copy(v_hbm, vv_v, sem_in)
        halves = jnp.full((LANES,), 0.5, dtype=jnp.float32)
        ones = jnp.full((LANES,), 1.0, dtype=jnp.float32)
        zeros = jnp.zeros((LANES,), dtype=jnp.float32)
        for k in range(BPW // LANES):
            t, off = divmod(k * LANES, 128)
            half_v[t, 0, pl.ds(off, LANES)] = halves
            half_v[t, 1, pl.ds(off, LANES)] = halves
            ones_v[pl.ds(k * LANES, LANES)] = ones
            zero_v[pl.ds(k * LANES, LANES)] = zeros
        blk = pl.ds(wid * BLK_PER_W, BLK_PER_W)
        h_half = pltpu.async_copy(half_v, half_hbm.at[blk], sem_out)
        h_ones = pltpu.async_copy(ones_v, ones_hbm.at[pl.ds(base, BPW)], sem_out)
        h_zero = pltpu.async_copy(zero_v, zero_hbm.at[pl.ds(base, BPW)], sem_out)
        h_act.wait()
        h_dirs.wait()
        h_v.wait()
        lane = lax.iota(jnp.int32, LANES)
        row = lane // 2
        col = lane - row * 2
        tab_v[...] = (plsc.load_gather(dirs_v, [col, row])
                      + plsc.load_gather(vv_v, [col]))

        def body(t, _):
            for u in range(8):
                k = t * 8 + u
                off = u * LANES
                a = idx_v[pl.ds(k * LANES, LANES)]
                i0 = a * 2
                g0 = plsc.load_gather(tab_v, [i0])
                g1 = plsc.load_gather(tab_v, [i0 + 1])
                out_v[t, 0, pl.ds(off, LANES)] = g0
                out_v[t, 1, pl.ds(off, LANES)] = g1
            return 0

        lax.fori_loop(0, BLK_PER_W, body, 0)
        h_out = pltpu.async_copy(out_v, out_hbm.at[blk], sem_out)
        h_half.wait()
        h_ones.wait()
        h_zero.wait()
        h_out.wait()

    return gather_add


_gather_add = _make_gather_kernel()


def kernel(state, context, action, v, dirs):
    act_flat = action.reshape(B)
    out_blocks, half2, ones1, zero1 = _gather_add(dirs.T, v, act_flat)
    delta = out_blocks.transpose(0, 2, 1).reshape(B, 1, 2)
    halves = half2.transpose(0, 2, 1).reshape(B, 1, 2)
    weights = ones1.reshape(B, 1)
    resid = zero1
    return ((delta, halves), weights, resid)


# speedup vs baseline: 3.6151x; 1.0545x over previous
import functools

import jax
import jax.numpy as jnp
from jax import lax
from jax.experimental import pallas as pl
from jax.experimental.pallas import tpu as pltpu
from jax.experimental.pallas import tpu_sc as plsc

B = 16384
NC = 1
NS = 16
NW = NC * NS
BPW = B // NW
LANES = 16
NBLK = B // 128
BLK_PER_W = BPW // 128


def _make_gather_kernel():
    mesh = plsc.VectorSubcoreMesh(core_axis_name="c", subcore_axis_name="s",
                                  num_cores=NC)

    @functools.partial(
        pl.kernel,
        mesh=mesh,
        compiler_params=pltpu.CompilerParams(needs_layout_passes=False),
        out_type=[
            jax.ShapeDtypeStruct((NBLK, 2, 128), jnp.float32),
            jax.ShapeDtypeStruct((NBLK, 2, 128), jnp.float32),
            jax.ShapeDtypeStruct((B,), jnp.float32),
            jax.ShapeDtypeStruct((B,), jnp.float32),
        ],
        scratch_types=[
            pltpu.VMEM((2, 4), jnp.float32),
            pltpu.VMEM((2,), jnp.float32),
            pltpu.VMEM((LANES,), jnp.float32),
            pltpu.VMEM((BPW,), jnp.int32),
            pltpu.VMEM((BLK_PER_W, 2, 128), jnp.float32),
            pltpu.VMEM((BLK_PER_W, 2, 128), jnp.float32),
            pltpu.VMEM((BPW,), jnp.float32),
            pltpu.VMEM((BPW,), jnp.float32),
            pltpu.SemaphoreType.DMA,
            pltpu.SemaphoreType.DMA,
        ],
    )
    def gather_add(dirs_hbm, v_hbm, act_hbm,
                   out_hbm, half_hbm, ones_hbm, zero_hbm,
                   dirs_v, vv_v, tab_v, idx_v, out_v, half_v, ones_v, zero_v,
                   sem_in, sem_out):
        wid = lax.axis_index("s") * NC + lax.axis_index("c")
        base = wid * BPW
        h_act = pltpu.async_copy(act_hbm.at[pl.ds(base, BPW)], idx_v, sem_in)
        h_dirs = pltpu.async_copy(dirs_hbm, dirs_v, sem_in)
        h_v = pltpu.async_copy(v_hbm, vv_v, sem_in)
        halves = jnp.full((LANES,), 0.5, dtype=jnp.float32)
        ones = jnp.full((LANES,), 1.0, dtype=jnp.float32)
        zeros = jnp.zeros((LANES,), dtype=jnp.float32)
        for k in range(BPW // LANES):
            t, off = divmod(k * LANES, 128)
            half_v[t, 0, pl.ds(off, LANES)] = halves
            half_v[t, 1, pl.ds(off, LANES)] = halves
            ones_v[pl.ds(k * LANES, LANES)] = ones
            zero_v[pl.ds(k * LANES, LANES)] = zeros
        blk = pl.ds(wid * BLK_PER_W, BLK_PER_W)
        h_half = pltpu.async_copy(half_v, half_hbm.at[blk], sem_out)
        h_ones = pltpu.async_copy(ones_v, ones_hbm.at[pl.ds(base, BPW)], sem_out)
        h_zero = pltpu.async_copy(zero_v, zero_hbm.at[pl.ds(base, BPW)], sem_out)
        h_act.wait()
        h_dirs.wait()
        h_v.wait()
        lane = lax.iota(jnp.int32, LANES)
        row = lane // 2
        col = lane - row * 2
        tab_v[...] = (plsc.load_gather(dirs_v, [col, row])
                      + plsc.load_gather(vv_v, [col]))

        def body(t, _):
            for u in range(8):
                k = t * 8 + u
                off = u * LANES
                a = idx_v[pl.ds(k * LANES, LANES)]
                i0 = a * 2
                g0 = plsc.load_gather(tab_v, [i0])
                g1 = plsc.load_gather(tab_v, [i0 + 1])
                out_v[t, 0, pl.ds(off, LANES)] = g0
                out_v[t, 1, pl.ds(off, LANES)] = g1
            return 0

        lax.fori_loop(0, BLK_PER_W, body, 0)
        h_out = pltpu.async_copy(out_v, out_hbm.at[blk], sem_out)
        h_half.wait()
        h_ones.wait()
        h_zero.wait()
        h_out.wait()

    return gather_add


_gather_add = _make_gather_kernel()


def kernel(state, context, action, v, dirs):
    act_flat = action.reshape(B)
    out_blocks, half2, ones1, zero1 = _gather_add(dirs.T, v, act_flat)
    delta = out_blocks.transpose(0, 2, 1).reshape(B, 1, 2)
    halves = half2.transpose(0, 2, 1).reshape(B, 1, 2)
    weights = ones1.reshape(B, 1)
    resid = zero1
    return ((delta, halves), weights, resid)


# speedup vs baseline: 3.6982x; 1.0230x over previous
import functools

import jax
import jax.numpy as jnp
from jax import lax
from jax.experimental import pallas as pl
from jax.experimental.pallas import tpu as pltpu
from jax.experimental.pallas import tpu_sc as plsc

B = 16384
NC = 1
NS = 16
NW = NC * NS
BPW = B // NW
LANES = 16
NBLK = B // 128
BLK_PER_W = BPW // 128


def _make_gather_kernel():
    mesh = plsc.VectorSubcoreMesh(core_axis_name="c", subcore_axis_name="s",
                                  num_cores=NC)

    @functools.partial(
        pl.kernel,
        mesh=mesh,
        compiler_params=pltpu.CompilerParams(needs_layout_passes=False),
        out_type=[
            jax.ShapeDtypeStruct((NBLK, 2, 128), jnp.float32),
            jax.ShapeDtypeStruct((NBLK, 2, 128), jnp.float32),
            jax.ShapeDtypeStruct((B,), jnp.float32),
            jax.ShapeDtypeStruct((B,), jnp.float32),
        ],
        scratch_types=[
            pltpu.VMEM((2, 4), jnp.float32),
            pltpu.VMEM((2,), jnp.float32),
            pltpu.VMEM((LANES,), jnp.float32),
            pltpu.VMEM((BPW,), jnp.int32),
            pltpu.VMEM((BLK_PER_W, 2, 128), jnp.float32),
            pltpu.VMEM((BLK_PER_W, 2, 128), jnp.float32),
            pltpu.VMEM((BPW,), jnp.float32),
            pltpu.VMEM((BPW,), jnp.float32),
            pltpu.SemaphoreType.DMA,
            pltpu.SemaphoreType.DMA,
        ],
    )
    def gather_add(dirs_hbm, v_hbm, act_hbm,
                   out_hbm, half_hbm, ones_hbm, zero_hbm,
                   dirs_v, vv_v, tab_v, idx_v, out_v, half_v, ones_v, zero_v,
                   sem_in, sem_out):
        wid = lax.axis_index("s") * NC + lax.axis_index("c")
        base = wid * BPW
        h_act = pltpu.async_copy(act_hbm.at[pl.ds(base, BPW)], idx_v, sem_in)
        h_dirs = pltpu.async_copy(dirs_hbm, dirs_v, sem_in)
        h_v = pltpu.async_copy(v_hbm, vv_v, sem_in)
        halves = jnp.full((LANES,), 0.5, dtype=jnp.float32)
        ones = jnp.full((LANES,), 1.0, dtype=jnp.float32)
        zeros = jnp.zeros((LANES,), dtype=jnp.float32)

        def fill_body(k, _):
            t = k // 8
            off = (k - t * 8) * LANES
            half_v[t, 0, pl.ds(off, LANES)] = halves
            half_v[t, 1, pl.ds(off, LANES)] = halves
            ones_v[pl.ds(k * LANES, LANES)] = ones
            zero_v[pl.ds(k * LANES, LANES)] = zeros
            return 0

        lax.fori_loop(0, BPW // LANES, fill_body, 0)
        blk = pl.ds(wid * BLK_PER_W, BLK_PER_W)
        h_half = pltpu.async_copy(half_v, half_hbm.at[blk], sem_out)
        h_ones = pltpu.async_copy(ones_v, ones_hbm.at[pl.ds(base, BPW)], sem_out)
        h_zero = pltpu.async_copy(zero_v, zero_hbm.at[pl.ds(base, BPW)], sem_out)
        h_act.wait()
        h_dirs.wait()
        h_v.wait()
        lane = lax.iota(jnp.int32, LANES)
        row = lane // 2
        col = lane - row * 2
        tab_v[...] = (plsc.load_gather(dirs_v, [col, row])
                      + plsc.load_gather(vv_v, [col]))

        def body(k, _):
            t = k // 8
            off = (k - t * 8) * LANES
            a = idx_v[pl.ds(k * LANES, LANES)]
            i0 = a * 2
            g0 = plsc.load_gather(tab_v, [i0])
            g1 = plsc.load_gather(tab_v, [i0 + 1])
            out_v[t, 0, pl.ds(off, LANES)] = g0
            out_v[t, 1, pl.ds(off, LANES)] = g1
            return 0

        lax.fori_loop(0, BPW // LANES, body, 0)
        h_out = pltpu.async_copy(out_v, out_hbm.at[blk], sem_out)
        h_half.wait()
        h_ones.wait()
        h_zero.wait()
        h_out.wait()

    return gather_add


_gather_add = _make_gather_kernel()


def kernel(state, context, action, v, dirs):
    act_flat = action.reshape(B)
    out_blocks, half2, ones1, zero1 = _gather_add(dirs.T, v, act_flat)
    delta = out_blocks.transpose(0, 2, 1).reshape(B, 1, 2)
    halves = half2.transpose(0, 2, 1).reshape(B, 1, 2)
    weights = ones1.reshape(B, 1)
    resid = zero1
    return ((delta, halves), weights, resid)


# speedup vs baseline: 3.7060x; 1.0021x over previous
import functools

import jax
import jax.numpy as jnp
from jax import lax
from jax.experimental import pallas as pl
from jax.experimental.pallas import tpu as pltpu
from jax.experimental.pallas import tpu_sc as plsc

B = 16384
NC = 1
NS = 16
NW = NC * NS
BPW = B // NW
LANES = 16
NBLK = B // 128
BLK_PER_W = BPW // 128


def _make_gather_kernel():
    mesh = plsc.VectorSubcoreMesh(core_axis_name="c", subcore_axis_name="s",
                                  num_cores=NC)

    @functools.partial(
        pl.kernel,
        mesh=mesh,
        compiler_params=pltpu.CompilerParams(needs_layout_passes=False),
        out_type=[
            jax.ShapeDtypeStruct((NBLK, 2, 128), jnp.float32),
            jax.ShapeDtypeStruct((NBLK, 2, 128), jnp.float32),
            jax.ShapeDtypeStruct((B,), jnp.float32),
            jax.ShapeDtypeStruct((B,), jnp.float32),
        ],
        scratch_types=[
            pltpu.VMEM((2, 4), jnp.float32),
            pltpu.VMEM((2,), jnp.float32),
            pltpu.VMEM((LANES,), jnp.float32),
            pltpu.VMEM((BPW,), jnp.int32),
            pltpu.VMEM((BLK_PER_W, 2, 128), jnp.float32),
            pltpu.VMEM((BLK_PER_W, 2, 128), jnp.float32),
            pltpu.VMEM((BPW,), jnp.float32),
            pltpu.VMEM((BPW,), jnp.float32),
            pltpu.SemaphoreType.DMA,
            pltpu.SemaphoreType.DMA,
        ],
    )
    def gather_add(dirs_hbm, v_hbm, act_hbm,
                   out_hbm, half_hbm, ones_hbm, zero_hbm,
                   dirs_v, vv_v, tab_v, idx_v, out_v, half_v, ones_v, zero_v,
                   sem_in, sem_out):
        wid = lax.axis_index("s") * NC + lax.axis_index("c")
        base = wid * BPW
        h_act = pltpu.async_copy(act_hbm.at[pl.ds(base, BPW)], idx_v, sem_in)
        h_dirs = pltpu.async_copy(dirs_hbm, dirs_v, sem_in)
        h_v = pltpu.async_copy(v_hbm, vv_v, sem_in)
        halves = jnp.full((LANES,), 0.5, dtype=jnp.float32)
        ones = jnp.full((LANES,), 1.0, dtype=jnp.float32)
        zeros = jnp.zeros((LANES,), dtype=jnp.float32)
        h_act.wait()
        h_dirs.wait()
        h_v.wait()
        lane = lax.iota(jnp.int32, LANES)
        row = lane // 2
        col = lane - row * 2
        tab_v[...] = (plsc.load_gather(dirs_v, [col, row])
                      + plsc.load_gather(vv_v, [col]))

        def body(k, _):
            t = k // 8
            off = (k - t * 8) * LANES
            a = idx_v[pl.ds(k * LANES, LANES)]
            i0 = a * 2
            g0 = plsc.load_gather(tab_v, [i0])
            g1 = plsc.load_gather(tab_v, [i0 + 1])
            out_v[t, 0, pl.ds(off, LANES)] = g0
            out_v[t, 1, pl.ds(off, LANES)] = g1
            half_v[t, 0, pl.ds(off, LANES)] = halves
            half_v[t, 1, pl.ds(off, LANES)] = halves
            ones_v[pl.ds(k * LANES, LANES)] = ones
            zero_v[pl.ds(k * LANES, LANES)] = zeros
            return 0

        lax.fori_loop(0, BPW // LANES, body, 0)
        blk = pl.ds(wid * BLK_PER_W, BLK_PER_W)
        h_half = pltpu.async_copy(half_v, half_hbm.at[blk], sem_out)
        h_ones = pltpu.async_copy(ones_v, ones_hbm.at[pl.ds(base, BPW)], sem_out)
        h_zero = pltpu.async_copy(zero_v, zero_hbm.at[pl.ds(base, BPW)], sem_out)
        h_out = pltpu.async_copy(out_v, out_hbm.at[blk], sem_out)
        h_half.wait()
        h_ones.wait()
        h_zero.wait()
        h_out.wait()

    return gather_add


_gather_add = _make_gather_kernel()


def kernel(state, context, action, v, dirs):
    act_flat = action.reshape(B)
    out_blocks, half2, ones1, zero1 = _gather_add(dirs.T, v, act_flat)
    delta = out_blocks.transpose(0, 2, 1).reshape(B, 1, 2)
    halves = half2.transpose(0, 2, 1).reshape(B, 1, 2)
    weights = ones1.reshape(B, 1)
    resid = zero1
    return ((delta, halves), weights, resid)


# speedup vs baseline: 3.8096x; 1.0280x over previous
import functools

import jax
import jax.numpy as jnp
from jax import lax
from jax.experimental import pallas as pl
from jax.experimental.pallas import tpu as pltpu
from jax.experimental.pallas import tpu_sc as plsc

B = 16384
NC = 1
NS = 16
NW = NC * NS
BPW = B // NW
LANES = 16
NBLK = B // 128
BLK_PER_W = BPW // 128


def _make_gather_kernel():
    mesh = plsc.VectorSubcoreMesh(core_axis_name="c", subcore_axis_name="s",
                                  num_cores=NC)

    @functools.partial(
        pl.kernel,
        mesh=mesh,
        compiler_params=pltpu.CompilerParams(needs_layout_passes=False),
        out_type=[
            jax.ShapeDtypeStruct((NBLK, 2, 128), jnp.float32),
            jax.ShapeDtypeStruct((NBLK, 2, 128), jnp.float32),
            jax.ShapeDtypeStruct((B,), jnp.float32),
            jax.ShapeDtypeStruct((B,), jnp.float32),
        ],
        scratch_types=[
            pltpu.VMEM((2, 4), jnp.float32),
            pltpu.VMEM((2,), jnp.float32),
            pltpu.VMEM((LANES,), jnp.float32),
            pltpu.VMEM((BPW,), jnp.int32),
            pltpu.VMEM((BLK_PER_W, 2, 128), jnp.float32),
            pltpu.VMEM((BLK_PER_W, 2, 128), jnp.float32),
            pltpu.VMEM((BPW,), jnp.float32),
            pltpu.VMEM((BPW,), jnp.float32),
            pltpu.SemaphoreType.DMA,
            pltpu.SemaphoreType.DMA,
        ],
    )
    def gather_add(dirs_hbm, v_hbm, act_hbm,
                   out_hbm, half_hbm, ones_hbm, zero_hbm,
                   dirs_v, vv_v, tab_v, idx_v, out_v, half_v, ones_v, zero_v,
                   sem_in, sem_out):
        wid = lax.axis_index("s") * NC + lax.axis_index("c")
        base = wid * BPW
        h_act = pltpu.async_copy(act_hbm.at[pl.ds(base, BPW)], idx_v, sem_in)
        h_dirs = pltpu.async_copy(dirs_hbm, dirs_v, sem_in)
        h_v = pltpu.async_copy(v_hbm, vv_v, sem_in)
        halves = jnp.full((LANES,), 0.5, dtype=jnp.float32)
        ones = jnp.full((LANES,), 1.0, dtype=jnp.float32)
        zeros = jnp.zeros((LANES,), dtype=jnp.float32)
        h_act.wait()
        h_dirs.wait()
        h_v.wait()
        lane = lax.iota(jnp.int32, LANES)
        row = lane // 2
        col = lane - row * 2
        tab_v[...] = (plsc.load_gather(dirs_v, [col, row])
                      + plsc.load_gather(vv_v, [col]))

        @plsc.parallel_loop(0, BPW // LANES, unroll=2)
        def body(k):
            t = k // 8
            off = (k - t * 8) * LANES
            a = idx_v[pl.ds(k * LANES, LANES)]
            i0 = a * 2
            g0 = plsc.load_gather(tab_v, [i0])
            g1 = plsc.load_gather(tab_v, [i0 + 1])
            out_v[t, 0, pl.ds(off, LANES)] = g0
            out_v[t, 1, pl.ds(off, LANES)] = g1
            half_v[t, 0, pl.ds(off, LANES)] = halves
            half_v[t, 1, pl.ds(off, LANES)] = halves
            ones_v[pl.ds(k * LANES, LANES)] = ones
            zero_v[pl.ds(k * LANES, LANES)] = zeros
        blk = pl.ds(wid * BLK_PER_W, BLK_PER_W)
        h_half = pltpu.async_copy(half_v, half_hbm.at[blk], sem_out)
        h_ones = pltpu.async_copy(ones_v, ones_hbm.at[pl.ds(base, BPW)], sem_out)
        h_zero = pltpu.async_copy(zero_v, zero_hbm.at[pl.ds(base, BPW)], sem_out)
        h_out = pltpu.async_copy(out_v, out_hbm.at[blk], sem_out)
        h_half.wait()
        h_ones.wait()
        h_zero.wait()
        h_out.wait()

    return gather_add


_gather_add = _make_gather_kernel()


def kernel(state, context, action, v, dirs):
    act_flat = action.reshape(B)
    out_blocks, half2, ones1, zero1 = _gather_add(dirs.T, v, act_flat)
    delta = out_blocks.transpose(0, 2, 1).reshape(B, 1, 2)
    halves = half2.transpose(0, 2, 1).reshape(B, 1, 2)
    weights = ones1.reshape(B, 1)
    resid = zero1
    return ((delta, halves), weights, resid)
